# baseline (speedup 1.0000x reference)
import functools

import jax
import jax.numpy as jnp
from jax.experimental import pallas as pl
from jax.experimental.pallas import tpu as pltpu

NUM_LABEL = 64
TQ = 512


def _proto_kernel(xm_ref, ys_ref, mask_ref, xt_ref, out_ref, protos_ref):
    j = pl.program_id(1)

    @pl.when(j == 0)
    def _build_protos():
        xm = xm_ref[0]
        ys = ys_ref[0, 0]
        n_meta = ys.shape[0]
        labels = jax.lax.broadcasted_iota(jnp.int32, (n_meta, NUM_LABEL), 1)
        onehot = (ys[:, None] == labels).astype(jnp.float32)
        sums = jax.lax.dot_general(
            onehot, xm, (((0,), (0,)), ((), ())),
            preferred_element_type=jnp.float32)
        counts = jnp.sum(onehot, axis=0)
        protos_ref[...] = sums / jnp.maximum(counts, 1.0)[:, None]

    x = xt_ref[0]
    p = protos_ref[...]
    xn = jnp.sum(x * x, axis=1)
    pn = jnp.sum(p * p, axis=1)
    xp = jax.lax.dot_general(
        x.astype(jnp.bfloat16), p.astype(jnp.bfloat16),
        (((1,), (1,)), ((), ())),
        preferred_element_type=jnp.float32)
    d2 = jnp.maximum(xn[:, None] + pn[None, :] - 2.0 * xp, 0.0)
    dist = -jnp.sqrt(d2)
    m = jnp.max(dist, axis=1, keepdims=True)
    e = jnp.exp(dist - m)
    probs = e / jnp.sum(e, axis=1, keepdims=True)
    out_ref[0] = probs * mask_ref[...]


@functools.partial(jax.jit, static_argnames=())
def kernel(xs_targ, xs_meta, ys_meta, max_N_label):
    B, N_targ, d = xs_targ.shape
    N_meta = xs_meta.shape[1]
    nq = N_targ // TQ
    ys3 = ys_meta.reshape(B, 1, N_meta).astype(jnp.int32)
    label_mask = (jnp.arange(NUM_LABEL) < max_N_label).astype(
        jnp.float32).reshape(1, NUM_LABEL)

    out = pl.pallas_call(
        _proto_kernel,
        grid=(B, nq),
        in_specs=[
            pl.BlockSpec((1, N_meta, d), lambda b, j: (b, 0, 0)),
            pl.BlockSpec((1, 1, N_meta), lambda b, j: (b, 0, 0)),
            pl.BlockSpec((1, NUM_LABEL), lambda b, j: (0, 0)),
            pl.BlockSpec((1, TQ, d), lambda b, j: (b, j, 0)),
        ],
        out_specs=pl.BlockSpec((1, TQ, NUM_LABEL), lambda b, j: (b, j, 0)),
        out_shape=jax.ShapeDtypeStruct((B, N_targ, NUM_LABEL), jnp.float32),
        scratch_shapes=[pltpu.VMEM((NUM_LABEL, d), jnp.float32)],
        compiler_params=pltpu.CompilerParams(
            dimension_semantics=("arbitrary", "arbitrary")),
    )(xs_meta, ys3, label_mask, xs_targ)
    return out.reshape(B * N_targ, NUM_LABEL)


# speedup vs baseline: 1.2676x; 1.2676x over previous
import functools

import jax
import jax.numpy as jnp
from jax.experimental import pallas as pl
from jax.experimental.pallas import tpu as pltpu

NUM_LABEL = 64
TQ = 1024


def _proto_kernel(xm_ref, ys_ref, mask_ref, xt_ref, out_ref, protos_ref):
    j = pl.program_id(1)

    @pl.when(j == 0)
    def _build_protos():
        xm = xm_ref[0]
        ys = ys_ref[0, 0]
        n_meta = ys.shape[0]
        labels = jax.lax.broadcasted_iota(jnp.int32, (n_meta, NUM_LABEL), 1)
        onehot = (ys[:, None] == labels).astype(jnp.float32)
        sums = jax.lax.dot_general(
            onehot, xm, (((0,), (0,)), ((), ())),
            preferred_element_type=jnp.float32)
        counts = jnp.sum(onehot, axis=0)
        protos_ref[...] = sums / jnp.maximum(counts, 1.0)[:, None]

    x = xt_ref[0]
    p = protos_ref[...]
    xn = jnp.sum(x * x, axis=1)
    pn = jnp.sum(p * p, axis=1)
    xp = jax.lax.dot_general(
        x.astype(jnp.bfloat16), p.astype(jnp.bfloat16),
        (((1,), (1,)), ((), ())),
        preferred_element_type=jnp.float32)
    d2 = jnp.maximum(xn[:, None] + pn[None, :] - 2.0 * xp, 0.0)
    dist = -jnp.sqrt(d2)
    m = jnp.max(dist, axis=1, keepdims=True)
    e = jnp.exp(dist - m)
    probs = e / jnp.sum(e, axis=1, keepdims=True)
    out_ref[0] = probs * mask_ref[...]


@functools.partial(jax.jit, static_argnames=())
def kernel(xs_targ, xs_meta, ys_meta, max_N_label):
    B, N_targ, d = xs_targ.shape
    N_meta = xs_meta.shape[1]
    nq = N_targ // TQ
    ys3 = ys_meta.reshape(B, 1, N_meta).astype(jnp.int32)
    label_mask = (jnp.arange(NUM_LABEL) < max_N_label).astype(
        jnp.float32).reshape(1, NUM_LABEL)

    out = pl.pallas_call(
        _proto_kernel,
        grid=(B, nq),
        in_specs=[
            pl.BlockSpec((1, N_meta, d), lambda b, j: (b, 0, 0)),
            pl.BlockSpec((1, 1, N_meta), lambda b, j: (b, 0, 0)),
            pl.BlockSpec((1, NUM_LABEL), lambda b, j: (0, 0)),
            pl.BlockSpec((1, TQ, d), lambda b, j: (b, j, 0)),
        ],
        out_specs=pl.BlockSpec((1, TQ, NUM_LABEL), lambda b, j: (b, j, 0)),
        out_shape=jax.ShapeDtypeStruct((B, N_targ, NUM_LABEL), jnp.float32),
        scratch_shapes=[pltpu.VMEM((NUM_LABEL, d), jnp.float32)],
        compiler_params=pltpu.CompilerParams(
            dimension_semantics=("arbitrary", "arbitrary")),
    )(xs_meta, ys3, label_mask, xs_targ)
    return out.reshape(B * N_targ, NUM_LABEL)


# speedup vs baseline: 1.4694x; 1.1592x over previous
import functools

import jax
import jax.numpy as jnp
from jax.experimental import pallas as pl
from jax.experimental.pallas import tpu as pltpu

NUM_LABEL = 64
TQ = 2048


def _proto_kernel(xm_ref, ys_ref, mask_ref, xt_ref, out_ref, protos_ref):
    j = pl.program_id(1)

    @pl.when(j == 0)
    def _build_protos():
        xm = xm_ref[0]
        ys = ys_ref[0, 0]
        n_meta = ys.shape[0]
        labels = jax.lax.broadcasted_iota(jnp.int32, (n_meta, NUM_LABEL), 1)
        onehot = (ys[:, None] == labels).astype(jnp.float32)
        sums = jax.lax.dot_general(
            onehot, xm, (((0,), (0,)), ((), ())),
            preferred_element_type=jnp.float32)
        counts = jnp.sum(onehot, axis=0)
        protos_ref[...] = sums / jnp.maximum(counts, 1.0)[:, None]

    x = xt_ref[0]
    p = protos_ref[...]
    xn = jnp.sum(x * x, axis=1)
    pn = jnp.sum(p * p, axis=1)
    xp = jax.lax.dot_general(
        x.astype(jnp.bfloat16), p.astype(jnp.bfloat16),
        (((1,), (1,)), ((), ())),
        preferred_element_type=jnp.float32)
    d2 = jnp.maximum(xn[:, None] + pn[None, :] - 2.0 * xp, 0.0)
    dist = -jnp.sqrt(d2)
    m = jnp.max(dist, axis=1, keepdims=True)
    e = jnp.exp(dist - m)
    probs = e / jnp.sum(e, axis=1, keepdims=True)
    out_ref[0] = probs * mask_ref[...]


@functools.partial(jax.jit, static_argnames=())
def kernel(xs_targ, xs_meta, ys_meta, max_N_label):
    B, N_targ, d = xs_targ.shape
    N_meta = xs_meta.shape[1]
    nq = N_targ // TQ
    ys3 = ys_meta.reshape(B, 1, N_meta).astype(jnp.int32)
    label_mask = (jnp.arange(NUM_LABEL) < max_N_label).astype(
        jnp.float32).reshape(1, NUM_LABEL)

    out = pl.pallas_call(
        _proto_kernel,
        grid=(B, nq),
        in_specs=[
            pl.BlockSpec((1, N_meta, d), lambda b, j: (b, 0, 0)),
            pl.BlockSpec((1, 1, N_meta), lambda b, j: (b, 0, 0)),
            pl.BlockSpec((1, NUM_LABEL), lambda b, j: (0, 0)),
            pl.BlockSpec((1, TQ, d), lambda b, j: (b, j, 0)),
        ],
        out_specs=pl.BlockSpec((1, TQ, NUM_LABEL), lambda b, j: (b, j, 0)),
        out_shape=jax.ShapeDtypeStruct((B, N_targ, NUM_LABEL), jnp.float32),
        scratch_shapes=[pltpu.VMEM((NUM_LABEL, d), jnp.float32)],
        compiler_params=pltpu.CompilerParams(
            dimension_semantics=("arbitrary", "arbitrary")),
    )(xs_meta, ys3, label_mask, xs_targ)
    return out.reshape(B * N_targ, NUM_LABEL)


# speedup vs baseline: 1.7408x; 1.1847x over previous
import functools

import jax
import jax.numpy as jnp
from jax.experimental import pallas as pl
from jax.experimental.pallas import tpu as pltpu

NUM_LABEL = 64
TQ = 4096


def _proto_kernel(xm_ref, ys_ref, mask_ref, xt_ref, out_ref, protos_ref):
    j = pl.program_id(1)

    @pl.when(j == 0)
    def _build_protos():
        xm = xm_ref[0]
        ys = ys_ref[0, 0]
        n_meta = ys.shape[0]
        labels = jax.lax.broadcasted_iota(jnp.int32, (n_meta, NUM_LABEL), 1)
        onehot = (ys[:, None] == labels).astype(jnp.float32)
        sums = jax.lax.dot_general(
            onehot, xm, (((0,), (0,)), ((), ())),
            preferred_element_type=jnp.float32)
        counts = jnp.sum(onehot, axis=0)
        protos_ref[...] = sums / jnp.maximum(counts, 1.0)[:, None]

    x = xt_ref[0]
    p = protos_ref[...]
    xn = jnp.sum(x * x, axis=1)
    pn = jnp.sum(p * p, axis=1)
    xp = jax.lax.dot_general(
        x.astype(jnp.bfloat16), p.astype(jnp.bfloat16),
        (((1,), (1,)), ((), ())),
        preferred_element_type=jnp.float32)
    d2 = jnp.maximum(xn[:, None] + pn[None, :] - 2.0 * xp, 0.0)
    dist = -jnp.sqrt(d2)
    m = jnp.max(dist, axis=1, keepdims=True)
    e = jnp.exp(dist - m)
    probs = e / jnp.sum(e, axis=1, keepdims=True)
    out_ref[0] = probs * mask_ref[...]


@functools.partial(jax.jit, static_argnames=())
def kernel(xs_targ, xs_meta, ys_meta, max_N_label):
    B, N_targ, d = xs_targ.shape
    N_meta = xs_meta.shape[1]
    nq = N_targ // TQ
    ys3 = ys_meta.reshape(B, 1, N_meta).astype(jnp.int32)
    label_mask = (jnp.arange(NUM_LABEL) < max_N_label).astype(
        jnp.float32).reshape(1, NUM_LABEL)

    out = pl.pallas_call(
        _proto_kernel,
        grid=(B, nq),
        in_specs=[
            pl.BlockSpec((1, N_meta, d), lambda b, j: (b, 0, 0)),
            pl.BlockSpec((1, 1, N_meta), lambda b, j: (b, 0, 0)),
            pl.BlockSpec((1, NUM_LABEL), lambda b, j: (0, 0)),
            pl.BlockSpec((1, TQ, d), lambda b, j: (b, j, 0)),
        ],
        out_specs=pl.BlockSpec((1, TQ, NUM_LABEL), lambda b, j: (b, j, 0)),
        out_shape=jax.ShapeDtypeStruct((B, N_targ, NUM_LABEL), jnp.float32),
        scratch_shapes=[pltpu.VMEM((NUM_LABEL, d), jnp.float32)],
        compiler_params=pltpu.CompilerParams(
            dimension_semantics=("arbitrary", "arbitrary")),
    )(xs_meta, ys3, label_mask, xs_targ)
    return out.reshape(B * N_targ, NUM_LABEL)


# speedup vs baseline: 1.7458x; 1.0029x over previous
import jax
import jax.numpy as jnp
from jax.experimental import pallas as pl
from jax.experimental.pallas import tpu as pltpu

NUM_LABEL = 64


def _proto_kernel(xm_ref, ys_ref, mask_ref, xt_ref, out_ref):
    xm = xm_ref[0]
    ys = ys_ref[0, 0]
    n_meta, d = xm.shape
    labels = jax.lax.broadcasted_iota(jnp.int32, (n_meta, NUM_LABEL), 1)
    onehot = (ys[:, None] == labels).astype(jnp.float32)
    sums = jax.lax.dot_general(
        onehot, xm, (((0,), (0,)), ((), ())),
        preferred_element_type=jnp.float32)
    counts = jnp.sum(onehot, axis=0)
    protos = sums / jnp.maximum(counts, 1.0)[:, None]
    pn = jnp.sum(protos * protos, axis=1)
    pm2 = -2.0 * protos

    x = xt_ref[0]
    xp = jax.lax.dot_general(
        x, pm2, (((1,), (1,)), ((), ())),
        preferred_element_type=jnp.float32)
    xn = jax.lax.dot_general(
        x * x, jnp.ones((NUM_LABEL, d), jnp.float32),
        (((1,), (1,)), ((), ())),
        preferred_element_type=jnp.float32)
    d2 = jnp.maximum(xp + xn + pn[None, :], 0.0)
    e = jnp.exp(-jnp.sqrt(d2))
    s = jnp.sum(e, axis=1, keepdims=True)
    out_ref[0] = e * (1.0 / s) * mask_ref[...]


def kernel(xs_targ, xs_meta, ys_meta, max_N_label):
    B, N_targ, d = xs_targ.shape
    N_meta = xs_meta.shape[1]
    ys3 = ys_meta.reshape(B, 1, N_meta).astype(jnp.int32)
    label_mask = (jnp.arange(NUM_LABEL) < max_N_label).astype(
        jnp.float32).reshape(1, NUM_LABEL)

    out = pl.pallas_call(
        _proto_kernel,
        grid=(B,),
        in_specs=[
            pl.BlockSpec((1, N_meta, d), lambda b: (b, 0, 0)),
            pl.BlockSpec((1, 1, N_meta), lambda b: (b, 0, 0)),
            pl.BlockSpec((1, NUM_LABEL), lambda b: (0, 0)),
            pl.BlockSpec((1, N_targ, d), lambda b: (b, 0, 0)),
        ],
        out_specs=pl.BlockSpec((1, N_targ, NUM_LABEL), lambda b: (b, 0, 0)),
        out_shape=jax.ShapeDtypeStruct((B, N_targ, NUM_LABEL), jnp.float32),
        compiler_params=pltpu.CompilerParams(
            dimension_semantics=("arbitrary",)),
    )(xs_meta, ys3, label_mask, xs_targ)
    return out.reshape(B * N_targ, NUM_LABEL)


# speedup vs baseline: 2.6124x; 1.4963x over previous
import jax
import jax.numpy as jnp
from jax.experimental import pallas as pl
from jax.experimental.pallas import tpu as pltpu

NUM_LABEL = 64


def _proto_kernel(xm_ref, ys_ref, mask_ref, xt_ref, out_ref):
    xm = xm_ref[0]
    ys = ys_ref[0, 0]
    n_meta, d = xm.shape
    labels = jax.lax.broadcasted_iota(jnp.int32, (n_meta, NUM_LABEL), 1)
    onehot = (ys[:, None] == labels).astype(jnp.float32)
    sums = jax.lax.dot_general(
        onehot, xm, (((0,), (0,)), ((), ())),
        preferred_element_type=jnp.float32)
    counts = jnp.sum(onehot, axis=0)
    protos = sums / jnp.maximum(counts, 1.0)[:, None]
    pn = jnp.sum(protos * protos, axis=1)
    pm2 = -2.0 * protos

    x = xt_ref[0]
    xpT = jax.lax.dot_general(
        pm2, x, (((1,), (1,)), ((), ())),
        preferred_element_type=jnp.float32)
    xnT = jax.lax.dot_general(
        jnp.ones((NUM_LABEL, d), jnp.float32), x * x,
        (((1,), (1,)), ((), ())),
        preferred_element_type=jnp.float32)
    d2 = jnp.maximum(xpT + xnT + pn[:, None], 0.0)
    e = jnp.exp(-jnp.sqrt(d2))
    s = jnp.sum(e, axis=0, keepdims=True)
    mT = jnp.transpose(mask_ref[...], (1, 0))
    out_ref[...] = e * (1.0 / s) * mT


def kernel(xs_targ, xs_meta, ys_meta, max_N_label):
    B, N_targ, d = xs_targ.shape
    N_meta = xs_meta.shape[1]
    ys3 = ys_meta.reshape(B, 1, N_meta).astype(jnp.int32)
    label_mask = (jnp.arange(NUM_LABEL) < max_N_label).astype(
        jnp.float32).reshape(1, NUM_LABEL)

    out = pl.pallas_call(
        _proto_kernel,
        grid=(B,),
        in_specs=[
            pl.BlockSpec((1, N_meta, d), lambda b: (b, 0, 0)),
            pl.BlockSpec((1, 1, N_meta), lambda b: (b, 0, 0)),
            pl.BlockSpec((1, NUM_LABEL), lambda b: (0, 0)),
            pl.BlockSpec((1, N_targ, d), lambda b: (b, 0, 0)),
        ],
        out_specs=pl.BlockSpec((NUM_LABEL, N_targ), lambda b: (0, b)),
        out_shape=jax.ShapeDtypeStruct((NUM_LABEL, B * N_targ), jnp.float32),
        compiler_params=pltpu.CompilerParams(
            dimension_semantics=("arbitrary",)),
    )(xs_meta, ys3, label_mask, xs_targ)
    return out.T


# speedup vs baseline: 2.6484x; 1.0138x over previous
import jax
import jax.numpy as jnp
from jax.experimental import pallas as pl
from jax.experimental.pallas import tpu as pltpu

NUM_LABEL = 64


def _proto_kernel(xm_ref, ys_ref, mask_ref, xt_ref, out_ref):
    xm = xm_ref[0]
    ys = ys_ref[0, 0]
    n_meta, d = xm.shape
    labels = jax.lax.broadcasted_iota(jnp.int32, (n_meta, NUM_LABEL), 1)
    onehot = (ys[:, None] == labels).astype(jnp.float32)
    sums = jax.lax.dot_general(
        onehot, xm, (((0,), (0,)), ((), ())),
        preferred_element_type=jnp.float32)
    counts = jnp.sum(onehot, axis=0)
    protos = sums / jnp.maximum(counts, 1.0)[:, None]
    pn = jnp.sum(protos * protos, axis=1)
    pm2 = -2.0 * protos

    x = xt_ref[0]
    xpT = jax.lax.dot_general(
        pm2.astype(jnp.bfloat16), x.astype(jnp.bfloat16),
        (((1,), (1,)), ((), ())),
        preferred_element_type=jnp.float32)
    xn8 = jax.lax.dot_general(
        jnp.ones((8, d), jnp.float32), x * x,
        (((1,), (1,)), ((), ())),
        preferred_element_type=jnp.float32)
    d2 = jnp.maximum(xpT + xn8[0:1, :] + pn[:, None], 0.0)
    e = jnp.exp(-jnp.sqrt(d2))
    s = jnp.sum(e, axis=0, keepdims=True)
    mT = jnp.transpose(mask_ref[...], (1, 0))
    out_ref[...] = e * (1.0 / s) * mT


def kernel(xs_targ, xs_meta, ys_meta, max_N_label):
    B, N_targ, d = xs_targ.shape
    N_meta = xs_meta.shape[1]
    ys3 = ys_meta.reshape(B, 1, N_meta).astype(jnp.int32)
    label_mask = (jnp.arange(NUM_LABEL) < max_N_label).astype(
        jnp.float32).reshape(1, NUM_LABEL)

    out = pl.pallas_call(
        _proto_kernel,
        grid=(B,),
        in_specs=[
            pl.BlockSpec((1, N_meta, d), lambda b: (b, 0, 0)),
            pl.BlockSpec((1, 1, N_meta), lambda b: (b, 0, 0)),
            pl.BlockSpec((1, NUM_LABEL), lambda b: (0, 0)),
            pl.BlockSpec((1, N_targ, d), lambda b: (b, 0, 0)),
        ],
        out_specs=pl.BlockSpec((NUM_LABEL, N_targ), lambda b: (0, b)),
        out_shape=jax.ShapeDtypeStruct((NUM_LABEL, B * N_targ), jnp.float32),
        compiler_params=pltpu.CompilerParams(
            dimension_semantics=("arbitrary",)),
    )(xs_meta, ys3, label_mask, xs_targ)
    return out.T
